# baseline (speedup 1.0000x reference)
import functools

import jax
import jax.numpy as jnp
from jax import lax
from jax.experimental import pallas as pl
from jax.experimental.pallas import tpu as pltpu
from jax.experimental.pallas import tpu_sc as plsc

N_GRAPHS = 3
N_NODES = 10000
N_EDGES = 320000
N_IN = 50
N_H = 128
N_SAMPLES = 2
VARIANCE_WEIGHT = 0.1
N_TUPLES = 8192
TUPLE_W = 7

NC, NS, L = 2, 16, 16
NW = NC * NS
EPW = N_EDGES // NW
CHUNK = 80
NCHUNK = EPW // CHUNK
NPAD = 10240
RPT = NPAD // NS
ZR = 32


def _make_segsum(n_pass, n_deg):
    width = N_H
    NB = 4
    mesh = plsc.VectorSubcoreMesh(core_axis_name="c", subcore_axis_name="s")

    @functools.partial(
        pl.kernel,
        out_type=jax.ShapeDtypeStruct((n_pass + n_deg, NC, NPAD, width),
                                      jnp.float32),
        mesh=mesh,
        scratch_types=[
            [pltpu.VMEM((3 * CHUNK,), jnp.int32)] * NB,
            [pltpu.VMEM((CHUNK,), jnp.int32)] * NB,
            [pltpu.VMEM((CHUNK,), jnp.int32)] * NB,
            [pltpu.VMEM((CHUNK, N_H), jnp.float32)] * NB,
            pltpu.VMEM((ZR, N_H), jnp.float32),
            pltpu.VMEM_SHARED((NPAD, N_H), jnp.float32),
            [pltpu.SemaphoreType.DMA] * NB,
            [pltpu.SemaphoreType.DMA] * NB,
            [pltpu.SemaphoreType.DMA] * NB,
        ],
    )
    def seg_kernel(tbl, edata, out, ebuf, sidx, didx, rows, zbuf,
                   acc, isems, gsems, ssems):
        cid = lax.axis_index("c")
        sid = lax.axis_index("s")
        wid = sid * NC + cid

        def zfill(r, _):
            for v in range(width // L):
                zbuf[r, pl.ds(v * L, L)] = jnp.zeros((L,), jnp.float32)
            return 0

        lax.fori_loop(0, ZR, zfill, 0)
        for z in range(RPT // ZR):
            pltpu.sync_copy(zbuf, acc.at[pl.ds(sid * RPT + z * ZR, ZR)])
        plsc.subcore_barrier()

        def issue_stage(i, b, slab):
            base = ((slab * NW + wid) * NCHUNK + i) * (3 * CHUNK)
            pltpu.async_copy(edata.at[pl.ds(base, 3 * CHUNK)], ebuf[b],
                             isems[b])

        def wait_stage(b):
            pltpu.make_async_copy(edata.at[pl.ds(0, 3 * CHUNK)], ebuf[b],
                                  isems[b]).wait()

        def stage_didx(b):
            for k in range(CHUNK // L):
                didx[b][pl.ds(k * L, L)] = ebuf[b][pl.ds(CHUNK + k * L, L)]

        def issue_gather(b):
            for k in range(CHUNK // L):
                sidx[b][pl.ds(k * L, L)] = ebuf[b][pl.ds(k * L, L)]
            pltpu.async_copy(tbl.at[sidx[b]], rows[b], gsems[b])

        def wait_gather(b):
            pltpu.make_async_copy(tbl.at[sidx[b]], rows[b], gsems[b]).wait()

        def issue_scatter(b):
            pltpu.async_copy(rows[b], acc.at[didx[b]], ssems[b], add=True)

        def wait_scatter(b):
            pltpu.make_async_copy(rows[b], acc.at[didx[b]], ssems[b]).wait()

        def scale(b):
            for j in range(CHUNK):
                if j % L == 0:
                    wv = lax.bitcast_convert_type(
                        ebuf[b][pl.ds(2 * CHUNK + j, L)], jnp.float32)
                wj = wv[j % L]
                for v in range(width // L):
                    rows[b][j, pl.ds(v * L, L)] = (
                        rows[b][j, pl.ds(v * L, L)] * wj)

        def build(b):
            for j in range(CHUNK):
                if j % L == 0:
                    wv = lax.bitcast_convert_type(
                        ebuf[b][pl.ds(2 * CHUNK + j, L)], jnp.float32)
                rows[b][j, pl.ds(0, L)] = jnp.broadcast_to(wv[j % L], (L,))

        def readout_and_rezero(p):
            plsc.subcore_barrier()
            pltpu.sync_copy(acc.at[pl.ds(sid * RPT, RPT)],
                            out.at[p, cid, pl.ds(sid * RPT, RPT)])
            for z in range(RPT // ZR):
                pltpu.sync_copy(zbuf, acc.at[pl.ds(sid * RPT + z * ZR, ZR)])
            plsc.subcore_barrier()

        def run_pass(p, slab, gathered):
            issue_stage(0, 0, slab)
            issue_stage(1, 1, slab)
            if gathered:
                wait_stage(0)
                issue_gather(0)

            def ring(ii, _):
                for t in range(NB):
                    i = NB * ii + t
                    b = t
                    bs = (t + 2) % NB
                    bg = (t + 1) % NB

                    @pl.when(i < NCHUNK)
                    def _():
                        @pl.when(i + 2 < NCHUNK)
                        def _():
                            @pl.when(i >= 2)
                            def _():
                                wait_scatter(bs)
                            issue_stage(i + 2, bs, slab)

                        if gathered:
                            @pl.when(i + 1 < NCHUNK)
                            def _():
                                wait_stage(bg)
                                issue_gather(bg)

                            stage_didx(b)
                            wait_gather(b)
                            scale(b)
                            issue_scatter(b)
                        else:
                            wait_stage(b)
                            build(b)
                            stage_didx(b)
                            issue_scatter(b)

                return 0

            lax.fori_loop(0, (NCHUNK + NB - 1) // NB, ring, 0)
            for t in range(NB):
                wait_scatter((NCHUNK - 1 - t) % NB)
            if not gathered:
                pass

        def one_pass(p, _):
            run_pass(p, p, True)
            readout_and_rezero(p)
            return 0

        lax.fori_loop(0, n_pass, one_pass, 0)

        def one_deg_pass(d, _):
            run_pass(n_pass + d, d, False)
            readout_and_rezero(n_pass + d)
            return 0

        if n_deg:
            for b in range(NB):
                def dzfill(j, _):
                    for v in range(1, width // L):
                        rows[b][j, pl.ds(v * L, L)] = jnp.zeros(
                            (L,), jnp.float32)
                    return 0

                lax.fori_loop(0, CHUNK, dzfill, 0)
            lax.fori_loop(0, n_deg, one_deg_pass, 0)

    return seg_kernel


TPW = N_TUPLES * TUPLE_W // NW
TCH = 128
NTCH = TPW // TCH


def _make_tuple_gather():
    mesh = plsc.VectorSubcoreMesh(core_axis_name="c", subcore_axis_name="s")

    @functools.partial(
        pl.kernel,
        out_type=jax.ShapeDtypeStruct(
            (N_GRAPHS, N_TUPLES * TUPLE_W, N_H), jnp.float32),
        mesh=mesh,
        scratch_types=[
            pltpu.VMEM((TCH,), jnp.int32),
            pltpu.VMEM((TCH, N_H), jnp.float32),
            pltpu.SemaphoreType.DMA,
        ],
    )
    def gather_kernel(tbl, idxf, out, tidx, rows, sem):
        cid = lax.axis_index("c")
        sid = lax.axis_index("s")
        wid = sid * NC + cid

        def one_pass(r, _):
            def one_chunk(i, _):
                base = wid * TPW + i * TCH
                pltpu.sync_copy(idxf.at[pl.ds(base, TCH)], tidx)
                off = r * (N_GRAPHS * N_NODES)
                for k in range(TCH // L):
                    tidx[pl.ds(k * L, L)] = tidx[pl.ds(k * L, L)] + off
                pltpu.async_copy(tbl.at[tidx], rows, sem).wait()
                pltpu.sync_copy(rows, out.at[r, pl.ds(base, TCH)])
                return 0

            lax.fori_loop(0, NTCH, one_chunk, 0)
            return 0

        lax.fori_loop(0, N_GRAPHS, one_pass, 0)

    return gather_kernel


def _stage_a_kernel(x_ref, mp_ref, w1_ref, out_ref):
    xm = x_ref[0] * mp_ref[0]
    out_ref[0] = jnp.dot(xm, w1_ref[...], preferred_element_type=jnp.float32)


def _stage_c_kernel(seg_ref, deg_ref, coff_ref, b1_ref, a1_ref, g1_ref,
                    bt1_ref, w2_ref, out_ref):
    seg = (seg_ref[0, 0] + seg_ref[0, 1])[:N_NODES]
    deg = (deg_ref[0, 0] + deg_ref[0, 1])[:N_NODES, 0:1]
    pre = seg + b1_ref[...] + deg * coff_ref[0, 0]
    h = jnp.where(pre > 0, pre, pre * a1_ref[...])
    mu = jnp.mean(h, axis=0, keepdims=True)
    var = jnp.mean((h - mu) ** 2, axis=0, keepdims=True)
    hb = (h - mu) / jnp.sqrt(var + 1e-5) * g1_ref[...] + bt1_ref[...]
    out_ref[0] = jnp.dot(hb, w2_ref[...], preferred_element_type=jnp.float32)


def _stage_e_kernel(seg_ref, b2_ref, a2_ref, g2_ref, bt2_ref, out_ref):
    pre = (seg_ref[0, 0] + seg_ref[0, 1])[:N_NODES] + b2_ref[...]
    h = jnp.where(pre > 0, pre, pre * a2_ref[...])
    mu = jnp.mean(h, axis=0, keepdims=True)
    var = jnp.mean((h - mu) ** 2, axis=0, keepdims=True)
    hb = (h - mu) / jnp.sqrt(var + 1e-5) * g2_ref[...] + bt2_ref[...]
    out_ref[0] = jnp.where(hb > 0, hb, jnp.exp(hb) - 1.0)


LOSS_BJ = 1024


def _loss_kernel(hi_ref, ht_ref, out_ref):
    j = pl.program_id(1)

    @pl.when(j == 0)
    def _():
        out_ref[...] = jnp.zeros_like(out_ref)

    hi = hi_ref[0]
    ht = ht_ref[0]
    dot = jnp.sum(hi[:, None, :] * ht, axis=2)
    ni = jnp.sqrt(jnp.sum(hi * hi, axis=1))[:, None]
    nt = jnp.sqrt(jnp.sum(ht * ht, axis=2))
    sim = dot / jnp.maximum(ni * nt, 1e-8)
    den = jnp.sum(jnp.exp(sim[:, 1:]), axis=1)
    part = jnp.sum(jnp.log(den) - sim[:, 0]) / N_TUPLES
    out_ref[...] += jnp.full((1, 1, N_H), part, jnp.float32)


def _combine_kernel(l_ref, out_ref):
    l = l_ref[...]
    l0, l1, l2 = l[:, 0:1], l[:, 1:2], l[:, 2:3]
    m = (l1 + l2) * 0.5
    var = (l1 - m) ** 2 + (l2 - m) ** 2
    out_ref[...] = l0 + VARIANCE_WEIGHT * var


def kernel(seq_list, edge_index, edge_weight, negative_sample, masks_logits,
           W1, b1, a1, g1, beta1, W2, b2, a2, g2, beta2,
           sparse, msk, samp_bias1, samp_bias2):
    f32 = jnp.float32
    mask_prob = jax.nn.sigmoid(masks_logits)
    mp3 = mask_prob[:N_GRAPHS]

    nkey = jax.random.key(42)
    noise = jnp.stack([
        jax.random.normal(jax.random.fold_in(nkey, i), (N_IN,), f32)
        * (1.0 - mask_prob[i])
        for i in range(N_SAMPLES)
    ])
    v = noise[:, None, :] * (1.0 - mp3)[None, :, :]
    c = jnp.dot(v.reshape(-1, N_IN), W1,
                precision=lax.Precision.HIGHEST)
    coff = jnp.concatenate(
        [jnp.zeros((1, N_GRAPHS, N_H), f32),
         c.reshape(N_SAMPLES, N_GRAPHS, N_H)], axis=0)

    src_r = edge_index[:, 1].reshape(
        N_GRAPHS, NW, NCHUNK, CHUNK).astype(jnp.int32)
    dst_r = edge_index[:, 0].reshape(
        N_GRAPHS, NW, NCHUNK, CHUNK).astype(jnp.int32)
    w_r = lax.bitcast_convert_type(
        edge_weight.astype(f32), jnp.int32).reshape(
            N_GRAPHS, NW, NCHUNK, CHUNK)

    def make_edata(n_pass):
        gsel = jnp.arange(n_pass, dtype=jnp.int32) % N_GRAPHS
        soff = (jnp.arange(n_pass, dtype=jnp.int32)
                * N_NODES)[:, None, None, None]
        return jnp.stack(
            [src_r[gsel] + soff, dst_r[gsel],
             w_r[gsel]], axis=3).reshape(-1)

    edata1 = make_edata(N_GRAPHS)
    edata2 = make_edata((N_SAMPLES + 1) * N_GRAPHS)
    tuples = negative_sample.reshape(-1).astype(jnp.int32)

    row128 = lambda x: x.reshape(1, N_H)

    h1aug = pl.pallas_call(
        _stage_a_kernel,
        grid=(N_GRAPHS,),
        in_specs=[
            pl.BlockSpec((1, N_NODES, N_IN), lambda g: (g, 0, 0)),
            pl.BlockSpec((1, 1, N_IN), lambda g: (g, 0, 0)),
            pl.BlockSpec((N_IN, N_H), lambda g: (0, 0)),
        ],
        out_specs=pl.BlockSpec((1, N_NODES, N_H), lambda g: (g, 0, 0)),
        out_shape=jax.ShapeDtypeStruct((N_GRAPHS, N_NODES, N_H), f32),
    )(seq_list, mp3.reshape(N_GRAPHS, 1, N_IN), W1)

    seg1 = _make_segsum(N_GRAPHS, N_GRAPHS)(
        h1aug.reshape(N_GRAPHS * N_NODES, N_H), edata1)

    h2 = pl.pallas_call(
        _stage_c_kernel,
        grid=(N_GRAPHS, N_SAMPLES + 1),
        in_specs=[
            pl.BlockSpec((1, NC, NPAD, N_H), lambda g, r: (g, 0, 0, 0)),
            pl.BlockSpec((1, NC, NPAD, N_H),
                         lambda g, r: (N_GRAPHS + g, 0, 0, 0)),
            pl.BlockSpec((1, 1, 1, N_H), lambda g, r: (r, g, 0, 0)),
            pl.BlockSpec((1, N_H), lambda g, r: (0, 0)),
            pl.BlockSpec((1, 1), lambda g, r: (0, 0)),
            pl.BlockSpec((1, N_H), lambda g, r: (0, 0)),
            pl.BlockSpec((1, N_H), lambda g, r: (0, 0)),
            pl.BlockSpec((N_H, N_H), lambda g, r: (0, 0)),
        ],
        out_specs=pl.BlockSpec((1, N_NODES, N_H),
                               lambda g, r: (r * N_GRAPHS + g, 0, 0)),
        out_shape=jax.ShapeDtypeStruct(
            ((N_SAMPLES + 1) * N_GRAPHS, N_NODES, N_H), f32),
    )(seg1, seg1, coff.reshape(N_GRAPHS, N_GRAPHS, 1, N_H), row128(b1),
      jnp.reshape(a1, (1, 1)), row128(g1), row128(beta1), W2)

    seg2 = _make_segsum((N_SAMPLES + 1) * N_GRAPHS, 0)(
        h2.reshape(-1, N_H), edata2)

    feats = pl.pallas_call(
        _stage_e_kernel,
        grid=((N_SAMPLES + 1) * N_GRAPHS,),
        in_specs=[
            pl.BlockSpec((1, NC, NPAD, N_H), lambda p: (p, 0, 0, 0)),
            pl.BlockSpec((1, N_H), lambda p: (0, 0)),
            pl.BlockSpec((1, 1), lambda p: (0, 0)),
            pl.BlockSpec((1, N_H), lambda p: (0, 0)),
            pl.BlockSpec((1, N_H), lambda p: (0, 0)),
        ],
        out_specs=pl.BlockSpec((1, N_NODES, N_H), lambda p: (p, 0, 0)),
        out_shape=jax.ShapeDtypeStruct(
            ((N_SAMPLES + 1) * N_GRAPHS, N_NODES, N_H), f32),
    )(seg2, row128(b2), jnp.reshape(a2, (1, 1)), row128(g2), row128(beta2))

    feats = feats.reshape(N_SAMPLES + 1, N_GRAPHS * N_NODES, N_H)

    h_t = _make_tuple_gather()(
        feats.reshape(-1, N_H), tuples)

    losses = pl.pallas_call(
        _loss_kernel,
        grid=(N_SAMPLES + 1, N_TUPLES // LOSS_BJ),
        in_specs=[
            pl.BlockSpec((1, LOSS_BJ, N_H), lambda r, j: (r, j, 0)),
            pl.BlockSpec((1, LOSS_BJ, TUPLE_W, N_H),
                         lambda r, j: (r, j, 0, 0)),
        ],
        out_specs=pl.BlockSpec((1, 1, N_H), lambda r, j: (r, 0, 0)),
        out_shape=jax.ShapeDtypeStruct((N_SAMPLES + 1, 1, N_H), f32),
    )(feats[:, :N_TUPLES], h_t.reshape(N_GRAPHS, N_TUPLES, TUPLE_W, N_H))

    out = pl.pallas_call(
        _combine_kernel,
        out_shape=jax.ShapeDtypeStruct((1, 1), f32),
    )(losses[:, 0, 0].reshape(1, 3))
    return out[0, 0]


# speedup vs baseline: 1.0459x; 1.0459x over previous
import functools

import jax
import jax.numpy as jnp
from jax import lax
from jax.experimental import pallas as pl
from jax.experimental.pallas import tpu as pltpu
from jax.experimental.pallas import tpu_sc as plsc

N_GRAPHS = 3
N_NODES = 10000
N_EDGES = 320000
N_IN = 50
N_H = 128
N_SAMPLES = 2
VARIANCE_WEIGHT = 0.1
N_TUPLES = 8192
TUPLE_W = 7

NC, NS, L = 2, 16, 16
NW = NC * NS
EPW = N_EDGES // NW
CHUNK = 80
NCHUNK = EPW // CHUNK
NPAD = 10240
RPT = NPAD // NS
ZR = 32


def _make_segsum(n_pass, n_deg):
    width = N_H
    NB = 4
    mesh = plsc.VectorSubcoreMesh(core_axis_name="c", subcore_axis_name="s")

    @functools.partial(
        pl.kernel,
        out_type=jax.ShapeDtypeStruct((n_pass + n_deg, NC, NPAD, width),
                                      jnp.float32),
        mesh=mesh,
        scratch_types=[
            [pltpu.VMEM((3 * CHUNK,), jnp.int32)] * NB,
            [pltpu.VMEM((CHUNK,), jnp.int32)] * NB,
            [pltpu.VMEM((CHUNK,), jnp.int32)] * NB,
            [pltpu.VMEM((CHUNK, N_H), jnp.float32)] * NB,
            pltpu.VMEM((ZR, N_H), jnp.float32),
            pltpu.VMEM_SHARED((NPAD, N_H), jnp.float32),
            [pltpu.SemaphoreType.DMA] * NB,
            [pltpu.SemaphoreType.DMA] * NB,
            [pltpu.SemaphoreType.DMA] * NB,
        ],
    )
    def seg_kernel(tbl, edata, out, ebuf, sidx, didx, rows, zbuf,
                   acc, isems, gsems, ssems):
        cid = lax.axis_index("c")
        sid = lax.axis_index("s")
        wid = sid * NC + cid

        def zfill(r, _):
            for v in range(width // L):
                zbuf[r, pl.ds(v * L, L)] = jnp.zeros((L,), jnp.float32)
            return 0

        lax.fori_loop(0, ZR, zfill, 0)
        for z in range(RPT // ZR):
            pltpu.sync_copy(zbuf, acc.at[pl.ds(sid * RPT + z * ZR, ZR)])
        plsc.subcore_barrier()

        def issue_stage(i, b, slab):
            base = ((slab * NW + wid) * NCHUNK + i) * (3 * CHUNK)
            pltpu.async_copy(edata.at[pl.ds(base, 3 * CHUNK)], ebuf[b],
                             isems[b])

        def wait_stage(b):
            pltpu.make_async_copy(edata.at[pl.ds(0, 3 * CHUNK)], ebuf[b],
                                  isems[b]).wait()

        def stage_didx(b):
            for k in range(CHUNK // L):
                didx[b][pl.ds(k * L, L)] = ebuf[b][pl.ds(CHUNK + k * L, L)]

        def issue_gather(b, row_off):
            for k in range(CHUNK // L):
                sidx[b][pl.ds(k * L, L)] = (
                    ebuf[b][pl.ds(k * L, L)] + row_off)
            pltpu.async_copy(tbl.at[sidx[b]], rows[b], gsems[b])

        def wait_gather(b):
            pltpu.make_async_copy(tbl.at[sidx[b]], rows[b], gsems[b]).wait()

        def issue_scatter(b):
            pltpu.async_copy(rows[b], acc.at[didx[b]], ssems[b], add=True)

        def wait_scatter(b):
            pltpu.make_async_copy(rows[b], acc.at[didx[b]], ssems[b]).wait()

        def scale(b):
            for j in range(CHUNK):
                if j % L == 0:
                    wv = lax.bitcast_convert_type(
                        ebuf[b][pl.ds(2 * CHUNK + j, L)], jnp.float32)
                wj = wv[j % L]
                for v in range(width // L):
                    rows[b][j, pl.ds(v * L, L)] = (
                        rows[b][j, pl.ds(v * L, L)] * wj)

        def build(b):
            for j in range(CHUNK):
                if j % L == 0:
                    wv = lax.bitcast_convert_type(
                        ebuf[b][pl.ds(2 * CHUNK + j, L)], jnp.float32)
                rows[b][j, pl.ds(0, L)] = jnp.broadcast_to(wv[j % L], (L,))

        def readout_and_rezero(p):
            plsc.subcore_barrier()
            pltpu.sync_copy(acc.at[pl.ds(sid * RPT, RPT)],
                            out.at[p, cid, pl.ds(sid * RPT, RPT)])
            for z in range(RPT // ZR):
                pltpu.sync_copy(zbuf, acc.at[pl.ds(sid * RPT + z * ZR, ZR)])
            plsc.subcore_barrier()

        def run_pass(p, g, row_off, gathered):
            issue_stage(0, 0, g)
            issue_stage(1, 1, g)
            if gathered:
                wait_stage(0)
                issue_gather(0, row_off)

            def ring(ii, _):
                for t in range(NB):
                    i = NB * ii + t
                    b = t
                    bs = (t + 2) % NB
                    bg = (t + 1) % NB

                    @pl.when(i < NCHUNK)
                    def _():
                        @pl.when(i + 2 < NCHUNK)
                        def _():
                            @pl.when(i >= 2)
                            def _():
                                wait_scatter(bs)
                            issue_stage(i + 2, bs, g)

                        if gathered:
                            @pl.when(i + 1 < NCHUNK)
                            def _():
                                wait_stage(bg)
                                issue_gather(bg, row_off)

                            stage_didx(b)
                            wait_gather(b)
                            scale(b)
                            issue_scatter(b)
                        else:
                            wait_stage(b)
                            build(b)
                            stage_didx(b)
                            issue_scatter(b)

                return 0

            lax.fori_loop(0, (NCHUNK + NB - 1) // NB, ring, 0)
            for t in range(NB):
                wait_scatter((NCHUNK - 1 - t) % NB)

        def one_pass(p, _):
            g = lax.rem(p, N_GRAPHS)
            run_pass(p, g, p * N_NODES, True)
            readout_and_rezero(p)
            return 0

        lax.fori_loop(0, n_pass, one_pass, 0)

        def one_deg_pass(d, _):
            run_pass(n_pass + d, d, 0, False)
            readout_and_rezero(n_pass + d)
            return 0

        if n_deg:
            for b in range(NB):
                def dzfill(j, _):
                    for v in range(1, width // L):
                        rows[b][j, pl.ds(v * L, L)] = jnp.zeros(
                            (L,), jnp.float32)
                    return 0

                lax.fori_loop(0, CHUNK, dzfill, 0)
            lax.fori_loop(0, n_deg, one_deg_pass, 0)

    return seg_kernel


TPW = N_TUPLES * TUPLE_W // NW
TCH = 128
NTCH = TPW // TCH


def _make_tuple_gather():
    mesh = plsc.VectorSubcoreMesh(core_axis_name="c", subcore_axis_name="s")

    @functools.partial(
        pl.kernel,
        out_type=jax.ShapeDtypeStruct(
            (N_GRAPHS, N_TUPLES * TUPLE_W, N_H), jnp.float32),
        mesh=mesh,
        scratch_types=[
            pltpu.VMEM((TCH,), jnp.int32),
            pltpu.VMEM((TCH, N_H), jnp.float32),
            pltpu.SemaphoreType.DMA,
        ],
    )
    def gather_kernel(tbl, idxf, out, tidx, rows, sem):
        cid = lax.axis_index("c")
        sid = lax.axis_index("s")
        wid = sid * NC + cid

        def one_pass(r, _):
            def one_chunk(i, _):
                base = wid * TPW + i * TCH
                pltpu.sync_copy(idxf.at[pl.ds(base, TCH)], tidx)
                off = r * (N_GRAPHS * N_NODES)
                for k in range(TCH // L):
                    tidx[pl.ds(k * L, L)] = tidx[pl.ds(k * L, L)] + off
                pltpu.async_copy(tbl.at[tidx], rows, sem).wait()
                pltpu.sync_copy(rows, out.at[r, pl.ds(base, TCH)])
                return 0

            lax.fori_loop(0, NTCH, one_chunk, 0)
            return 0

        lax.fori_loop(0, N_GRAPHS, one_pass, 0)

    return gather_kernel


def _stage_a_kernel(x_ref, mp_ref, w1_ref, out_ref):
    xm = x_ref[0] * mp_ref[0]
    out_ref[0] = jnp.dot(xm, w1_ref[...], preferred_element_type=jnp.float32)


def _stage_c_kernel(seg_ref, deg_ref, coff_ref, b1_ref, a1_ref, g1_ref,
                    bt1_ref, w2_ref, out_ref):
    seg = (seg_ref[0, 0] + seg_ref[0, 1])[:N_NODES]
    deg = (deg_ref[0, 0] + deg_ref[0, 1])[:N_NODES, 0:1]
    pre = seg + b1_ref[...] + deg * coff_ref[0, 0]
    h = jnp.where(pre > 0, pre, pre * a1_ref[...])
    mu = jnp.mean(h, axis=0, keepdims=True)
    var = jnp.mean((h - mu) ** 2, axis=0, keepdims=True)
    hb = (h - mu) / jnp.sqrt(var + 1e-5) * g1_ref[...] + bt1_ref[...]
    out_ref[0] = jnp.dot(hb, w2_ref[...], preferred_element_type=jnp.float32)


def _stage_e_kernel(seg_ref, b2_ref, a2_ref, g2_ref, bt2_ref, out_ref):
    pre = (seg_ref[0, 0] + seg_ref[0, 1])[:N_NODES] + b2_ref[...]
    h = jnp.where(pre > 0, pre, pre * a2_ref[...])
    mu = jnp.mean(h, axis=0, keepdims=True)
    var = jnp.mean((h - mu) ** 2, axis=0, keepdims=True)
    hb = (h - mu) / jnp.sqrt(var + 1e-5) * g2_ref[...] + bt2_ref[...]
    out_ref[0] = jnp.where(hb > 0, hb, jnp.exp(hb) - 1.0)


LOSS_BJ = 1024


def _loss_kernel(hi_ref, ht_ref, out_ref):
    j = pl.program_id(1)

    @pl.when(j == 0)
    def _():
        out_ref[...] = jnp.zeros_like(out_ref)

    hi = hi_ref[0]
    ht = ht_ref[0]
    dot = jnp.sum(hi[:, None, :] * ht, axis=2)
    ni = jnp.sqrt(jnp.sum(hi * hi, axis=1))[:, None]
    nt = jnp.sqrt(jnp.sum(ht * ht, axis=2))
    sim = dot / jnp.maximum(ni * nt, 1e-8)
    den = jnp.sum(jnp.exp(sim[:, 1:]), axis=1)
    part = jnp.sum(jnp.log(den) - sim[:, 0]) / N_TUPLES
    out_ref[...] += jnp.full((1, 1, N_H), part, jnp.float32)


def _combine_kernel(l_ref, out_ref):
    l = l_ref[...]
    l0, l1, l2 = l[:, 0:1], l[:, 1:2], l[:, 2:3]
    m = (l1 + l2) * 0.5
    var = (l1 - m) ** 2 + (l2 - m) ** 2
    out_ref[...] = l0 + VARIANCE_WEIGHT * var


def kernel(seq_list, edge_index, edge_weight, negative_sample, masks_logits,
           W1, b1, a1, g1, beta1, W2, b2, a2, g2, beta2,
           sparse, msk, samp_bias1, samp_bias2):
    f32 = jnp.float32
    mask_prob = jax.nn.sigmoid(masks_logits)
    mp3 = mask_prob[:N_GRAPHS]

    nkey = jax.random.key(42)
    noise = jnp.stack([
        jax.random.normal(jax.random.fold_in(nkey, i), (N_IN,), f32)
        * (1.0 - mask_prob[i])
        for i in range(N_SAMPLES)
    ])
    v = noise[:, None, :] * (1.0 - mp3)[None, :, :]
    c = jnp.dot(v.reshape(-1, N_IN), W1,
                precision=lax.Precision.HIGHEST)
    coff = jnp.concatenate(
        [jnp.zeros((1, N_GRAPHS, N_H), f32),
         c.reshape(N_SAMPLES, N_GRAPHS, N_H)], axis=0)

    src_r = edge_index[:, 1].reshape(
        N_GRAPHS, NW, NCHUNK, CHUNK).astype(jnp.int32)
    dst_r = edge_index[:, 0].reshape(
        N_GRAPHS, NW, NCHUNK, CHUNK).astype(jnp.int32)
    w_r = lax.bitcast_convert_type(
        edge_weight.astype(f32), jnp.int32).reshape(
            N_GRAPHS, NW, NCHUNK, CHUNK)

    edata = jnp.stack([src_r, dst_r, w_r], axis=3).reshape(-1)
    tuples = negative_sample.reshape(-1).astype(jnp.int32)

    row128 = lambda x: x.reshape(1, N_H)

    h1aug = pl.pallas_call(
        _stage_a_kernel,
        grid=(N_GRAPHS,),
        in_specs=[
            pl.BlockSpec((1, N_NODES, N_IN), lambda g: (g, 0, 0)),
            pl.BlockSpec((1, 1, N_IN), lambda g: (g, 0, 0)),
            pl.BlockSpec((N_IN, N_H), lambda g: (0, 0)),
        ],
        out_specs=pl.BlockSpec((1, N_NODES, N_H), lambda g: (g, 0, 0)),
        out_shape=jax.ShapeDtypeStruct((N_GRAPHS, N_NODES, N_H), f32),
    )(seq_list, mp3.reshape(N_GRAPHS, 1, N_IN), W1)

    seg1 = _make_segsum(N_GRAPHS, N_GRAPHS)(
        h1aug.reshape(N_GRAPHS * N_NODES, N_H), edata)

    h2 = pl.pallas_call(
        _stage_c_kernel,
        grid=(N_GRAPHS, N_SAMPLES + 1),
        in_specs=[
            pl.BlockSpec((1, NC, NPAD, N_H), lambda g, r: (g, 0, 0, 0)),
            pl.BlockSpec((1, NC, NPAD, N_H),
                         lambda g, r: (N_GRAPHS + g, 0, 0, 0)),
            pl.BlockSpec((1, 1, 1, N_H), lambda g, r: (r, g, 0, 0)),
            pl.BlockSpec((1, N_H), lambda g, r: (0, 0)),
            pl.BlockSpec((1, 1), lambda g, r: (0, 0)),
            pl.BlockSpec((1, N_H), lambda g, r: (0, 0)),
            pl.BlockSpec((1, N_H), lambda g, r: (0, 0)),
            pl.BlockSpec((N_H, N_H), lambda g, r: (0, 0)),
        ],
        out_specs=pl.BlockSpec((1, N_NODES, N_H),
                               lambda g, r: (r * N_GRAPHS + g, 0, 0)),
        out_shape=jax.ShapeDtypeStruct(
            ((N_SAMPLES + 1) * N_GRAPHS, N_NODES, N_H), f32),
    )(seg1, seg1, coff.reshape(N_GRAPHS, N_GRAPHS, 1, N_H), row128(b1),
      jnp.reshape(a1, (1, 1)), row128(g1), row128(beta1), W2)

    seg2 = _make_segsum((N_SAMPLES + 1) * N_GRAPHS, 0)(
        h2.reshape(-1, N_H), edata)

    feats = pl.pallas_call(
        _stage_e_kernel,
        grid=((N_SAMPLES + 1) * N_GRAPHS,),
        in_specs=[
            pl.BlockSpec((1, NC, NPAD, N_H), lambda p: (p, 0, 0, 0)),
            pl.BlockSpec((1, N_H), lambda p: (0, 0)),
            pl.BlockSpec((1, 1), lambda p: (0, 0)),
            pl.BlockSpec((1, N_H), lambda p: (0, 0)),
            pl.BlockSpec((1, N_H), lambda p: (0, 0)),
        ],
        out_specs=pl.BlockSpec((1, N_NODES, N_H), lambda p: (p, 0, 0)),
        out_shape=jax.ShapeDtypeStruct(
            ((N_SAMPLES + 1) * N_GRAPHS, N_NODES, N_H), f32),
    )(seg2, row128(b2), jnp.reshape(a2, (1, 1)), row128(g2), row128(beta2))

    feats = feats.reshape(N_SAMPLES + 1, N_GRAPHS * N_NODES, N_H)

    h_t = _make_tuple_gather()(
        feats.reshape(-1, N_H), tuples)

    losses = pl.pallas_call(
        _loss_kernel,
        grid=(N_SAMPLES + 1, N_TUPLES // LOSS_BJ),
        in_specs=[
            pl.BlockSpec((1, LOSS_BJ, N_H), lambda r, j: (r, j, 0)),
            pl.BlockSpec((1, LOSS_BJ, TUPLE_W, N_H),
                         lambda r, j: (r, j, 0, 0)),
        ],
        out_specs=pl.BlockSpec((1, 1, N_H), lambda r, j: (r, 0, 0)),
        out_shape=jax.ShapeDtypeStruct((N_SAMPLES + 1, 1, N_H), f32),
    )(feats[:, :N_TUPLES], h_t.reshape(N_GRAPHS, N_TUPLES, TUPLE_W, N_H))

    out = pl.pallas_call(
        _combine_kernel,
        out_shape=jax.ShapeDtypeStruct((1, 1), f32),
    )(losses[:, 0, 0].reshape(1, 3))
    return out[0, 0]
